# baseline (speedup 1.0000x reference)
import functools

import jax
import jax.numpy as jnp
from jax import lax
from jax.experimental import pallas as pl
from jax.experimental.pallas import tpu as pltpu
from jax.experimental.pallas import tpu_sc as plsc

_NC = 2
_NS = 16
_NW = _NC * _NS

_CW = 1024
_D = 64
_V = 1000000
_NFULL = _V // _CW
_TAIL = _V - _NFULL * _CW
_PER_T = _NFULL // _NW


def _sc_stream_extract(tableT, idx_sorted, pos_sorted, rng_w):
    b_total = idx_sorted.shape[0]
    mesh = plsc.VectorSubcoreMesh(core_axis_name="c", subcore_axis_name="s")

    @functools.partial(
        pl.kernel,
        mesh=mesh,
        out_type=jax.ShapeDtypeStruct(((b_total + _NW) * _D,), jnp.float32),
        compiler_params=pltpu.CompilerParams(needs_layout_passes=False),
        scratch_types=[
            pltpu.VMEM((b_total,), jnp.int32),
            pltpu.VMEM((b_total,), jnp.int32),
            pltpu.VMEM((_D, _CW), jnp.float32),
            pltpu.VMEM((16 * _D,), jnp.float32),
            pltpu.VMEM((16,), jnp.int32),
            pltpu.SemaphoreType.DMA,
        ],
    )
    def k(tab_hbm, idx_hbm, pos_hbm, rng_hbm, out_hbm,
          idx_v, pos_v, chunk_v, slots_v, rng_v, wsem):
        pltpu.sync_copy(idx_hbm, idx_v)
        pltpu.sync_copy(pos_hbm, pos_v)
        wid = lax.axis_index("s") * _NC + lax.axis_index("c")
        pltpu.sync_copy(rng_hbm.at[wid], rng_v)
        rv = rng_v[...]
        s_lo = rv[0]
        s_hi = rv[1]

        ri = [lax.iota(jnp.int32, 16) + 16 * m for m in range(4)]

        def process16(chunkid, iv, sel, pv):
            pv_eff = jnp.where(sel, pv, b_total + wid)

            @pl.when(jnp.any(sel))
            def _p():
                clv = jnp.clip(iv - chunkid * _CW, 0, _CW - 1)
                writes = []
                for j in range(16):
                    ci = jnp.broadcast_to(clv[j], (16,))
                    for m in range(4):
                        gat = plsc.load_gather(chunk_v, [ri[m], ci])
                        slots_v[pl.ds(j * _D + m * 16, 16)] = gat
                    writes.append(pltpu.async_copy(
                        slots_v.at[pl.ds(j * _D, _D)],
                        out_hbm.at[pl.ds(pl.multiple_of(pv_eff[j] * _D, _D), _D)],
                        wsem))
                for w in writes:
                    w.wait()

        def fetch(chunkid):
            @pl.when(chunkid >= _NFULL)
            def _t():
                pltpu.sync_copy(
                    tab_hbm.at[:, pl.ds(pl.multiple_of(chunkid * _CW, 128), 512)],
                    chunk_v.at[:, pl.ds(0, 512)])
                pltpu.sync_copy(
                    tab_hbm.at[:, pl.ds(pl.multiple_of(chunkid * _CW + 512, 128), 128)],
                    chunk_v.at[:, pl.ds(512, 128)])

            @pl.when(chunkid < _NFULL)
            def _f():
                pltpu.sync_copy(
                    tab_hbm.at[:, pl.ds(pl.multiple_of(chunkid * _CW, _CW), _CW)],
                    chunk_v)

        def grp(g, cur):
            iv = idx_v[pl.ds(g * 16, 16)]
            pv = pos_v[pl.ds(g * 16, 16)]
            e_vec = g * 16 + lax.iota(jnp.int32, 16)
            act = jnp.logical_and(e_vec >= s_lo, e_vec < s_hi)
            cv = jnp.where(act, iv // _CW, -1)
            sel = jnp.logical_and(act, cv == cur)
            process16(cur, iv, sel, pv)
            tgt = jnp.max(cv)

            def wbody(c):
                nxt = jnp.min(jnp.where(cv > c, cv, jnp.int32(1 << 30)))
                fetch(nxt)
                seln = jnp.logical_and(act, cv == nxt)
                process16(nxt, iv, seln, pv)
                return nxt

            return lax.while_loop(lambda c: c < tgt, wbody, cur)

        lax.fori_loop(s_lo // 16, (s_hi + 15) // 16, grp, jnp.int32(-1))

    return k(tableT, idx_sorted, pos_sorted, rng_w)


def _mlp_body(embT_ref, w1t_ref, b1_ref, w2t_ref, b2_ref, out_ref):
    x = jnp.dot(w1t_ref[...], embT_ref[...],
                preferred_element_type=jnp.float32)
    x = jnp.maximum(x + b1_ref[...], 0.0)
    out_ref[...] = jnp.sum(x * w2t_ref[...], axis=0, keepdims=True) + b2_ref[...]


def _tc_mlp_t(embT, W1T, b1, W2T, b2):
    d, b_total = embT.shape
    h = W1T.shape[0]
    blk = 2048
    grid = (b_total // blk,)
    return pl.pallas_call(
        _mlp_body,
        grid=grid,
        in_specs=[
            pl.BlockSpec((d, blk), lambda i: (0, i)),
            pl.BlockSpec((h, d), lambda i: (0, 0)),
            pl.BlockSpec((h, 1), lambda i: (0, 0)),
            pl.BlockSpec((h, 1), lambda i: (0, 0)),
            pl.BlockSpec((1, 1), lambda i: (0, 0)),
        ],
        out_specs=pl.BlockSpec((1, blk), lambda i: (0, i)),
        out_shape=jax.ShapeDtypeStruct((1, b_total), jnp.float32),
    )(embT, W1T, b1.reshape(h, 1), W2T.reshape(h, 1), b2.reshape(1, 1))


def kernel(states, emb_table, W1, b1, W2, b2):
    b_total = states.shape[0]
    d = emb_table.shape[1]
    idx1d = states.reshape(b_total)
    tableT = emb_table.T

    iota = lax.iota(jnp.int32, b_total)
    idx_sorted, pos_sorted = lax.sort_key_val(idx1d, iota)
    bnd = jnp.minimum(jnp.arange(_NW + 1) * (_PER_T * _CW), _V).astype(
        jnp.int32).at[_NW].set(_V)
    st = jnp.searchsorted(idx_sorted, bnd, side="left").astype(jnp.int32)
    rng_w = jnp.pad(jnp.stack([st[:-1], st[1:]], axis=1), ((0, 0), (0, 14)))

    flat = _sc_stream_extract(tableT, idx_sorted, pos_sorted, rng_w)
    emb = flat[: b_total * d].reshape(b_total, d)
    valT = _tc_mlp_t(emb.T, W1.T, b1, W2.reshape(-1), b2)
    return (emb, valT.T)


# speedup vs baseline: 1.1488x; 1.1488x over previous
import functools

import jax
import jax.numpy as jnp
from jax import lax
from jax.experimental import pallas as pl
from jax.experimental.pallas import tpu as pltpu
from jax.experimental.pallas import tpu_sc as plsc

_NC = 2
_NS = 16
_NW = _NC * _NS

_CW = 512
_D = 64
_V = 1000000
_NFULL = _V // _CW
_TAIL = _V - _NFULL * _CW
_PER_T = _NFULL // _NW


def _sc_stream_extract(tableT, idx_sorted, pos_sorted, rng_w):
    b_total = idx_sorted.shape[0]
    mesh = plsc.VectorSubcoreMesh(core_axis_name="c", subcore_axis_name="s")

    @functools.partial(
        pl.kernel,
        mesh=mesh,
        out_type=jax.ShapeDtypeStruct(((b_total + _NW) * _D,), jnp.float32),
        compiler_params=pltpu.CompilerParams(needs_layout_passes=False),
        scratch_types=[
            pltpu.VMEM((b_total,), jnp.int32),
            pltpu.VMEM((b_total,), jnp.int32),
            pltpu.VMEM((2, _D, _CW), jnp.float32),
            pltpu.VMEM((16 * _D,), jnp.float32),
            pltpu.VMEM((16,), jnp.int32),
            pltpu.SemaphoreType.DMA,
            pltpu.SemaphoreType.DMA,
        ],
    )
    def k(tab_hbm, idx_hbm, pos_hbm, rng_hbm, out_hbm,
          idx_v, pos_v, chunk_v, slots_v, rng_v, wsem, psem):
        pltpu.sync_copy(idx_hbm, idx_v)
        pltpu.sync_copy(pos_hbm, pos_v)
        wid = lax.axis_index("s") * _NC + lax.axis_index("c")
        pltpu.sync_copy(rng_hbm.at[wid], rng_v)
        rv = rng_v[...]
        s_lo = rv[0]
        s_hi = rv[1]

        ri = [lax.iota(jnp.int32, 16) + 16 * m for m in range(4)]

        def process16(chunkid, ab, iv, sel, pv):
            pv_eff = jnp.where(sel, pv, b_total + wid)

            @pl.when(jnp.any(sel))
            def _p():
                clv = jnp.clip(iv - chunkid * _CW, 0, _CW - 1)
                writes = []
                for j in range(16):
                    ci = jnp.broadcast_to(clv[j], (16,))
                    for m in range(4):
                        gat = plsc.load_gather(chunk_v.at[ab], [ri[m], ci])
                        slots_v[pl.ds(j * _D + m * 16, 16)] = gat
                    writes.append(pltpu.async_copy(
                        slots_v.at[pl.ds(j * _D, _D)],
                        out_hbm.at[pl.ds(pl.multiple_of(pv_eff[j] * _D, _D), _D)],
                        wsem))
                for w in writes:
                    w.wait()

        def fetch(chunkid, ab):
            @pl.when(chunkid >= _NFULL)
            def _t():
                pltpu.sync_copy(
                    tab_hbm.at[:, pl.ds(pl.multiple_of(chunkid * _CW, 128), 128)],
                    chunk_v.at[ab, :, pl.ds(0, 128)])

            @pl.when(chunkid < _NFULL)
            def _f():
                pltpu.sync_copy(
                    tab_hbm.at[:, pl.ds(pl.multiple_of(chunkid * _CW, _CW), _CW)],
                    chunk_v.at[ab])

        def prefetch(chunkid, ab):
            pltpu.async_copy(
                tab_hbm.at[:, pl.ds(pl.multiple_of(chunkid * _CW, _CW), _CW)],
                chunk_v.at[ab], psem)

        def drain_p():
            pltpu.make_async_copy(
                tab_hbm.at[:, pl.ds(0, _CW)], chunk_v.at[0], psem).wait()

        def grp(g, state):
            cur, ab, pend = state
            iv = idx_v[pl.ds(g * 16, 16)]
            pv = pos_v[pl.ds(g * 16, 16)]
            e_vec = g * 16 + lax.iota(jnp.int32, 16)
            act = jnp.logical_and(e_vec >= s_lo, e_vec < s_hi)
            cv = jnp.where(act, iv // _CW, -1)
            sel = jnp.logical_and(act, cv == cur)
            process16(cur, ab, iv, sel, pv)
            tgt = jnp.max(cv)

            def wbody(st):
                c, ab, pend = st
                nxt = jnp.min(jnp.where(cv > c, cv, jnp.int32(1 << 30)))
                hit = jnp.logical_and(pend == 1, nxt == c + 1)

                @pl.when(pend == 1)
                def _d():
                    drain_p()

                nab = 1 - ab

                @pl.when(jnp.logical_not(hit))
                def _f():
                    fetch(nxt, nab)

                can_pre = nxt + 1 < _NFULL

                @pl.when(can_pre)
                def _p():
                    prefetch(nxt + 1, ab)

                seln = jnp.logical_and(act, cv == nxt)
                process16(nxt, nab, iv, seln, pv)
                return (nxt, nab, jnp.where(can_pre, 1, 0).astype(jnp.int32))

            return lax.while_loop(lambda st: st[0] < tgt, wbody,
                                  (cur, ab, pend))

        fin = lax.fori_loop(
            s_lo // 16, (s_hi + 15) // 16, grp,
            (jnp.int32(-1), jnp.int32(0), jnp.int32(0)))

        @pl.when(fin[2] == 1)
        def _cleanup():
            drain_p()

    return k(tableT, idx_sorted, pos_sorted, rng_w)


def _mlp_body(embT_ref, w1t_ref, b1_ref, w2t_ref, b2_ref, out_ref):
    x = jnp.dot(w1t_ref[...], embT_ref[...],
                preferred_element_type=jnp.float32)
    x = jnp.maximum(x + b1_ref[...], 0.0)
    out_ref[...] = jnp.sum(x * w2t_ref[...], axis=0, keepdims=True) + b2_ref[...]


def _tc_mlp_t(embT, W1T, b1, W2T, b2):
    d, b_total = embT.shape
    h = W1T.shape[0]
    blk = 2048
    grid = (b_total // blk,)
    return pl.pallas_call(
        _mlp_body,
        grid=grid,
        in_specs=[
            pl.BlockSpec((d, blk), lambda i: (0, i)),
            pl.BlockSpec((h, d), lambda i: (0, 0)),
            pl.BlockSpec((h, 1), lambda i: (0, 0)),
            pl.BlockSpec((h, 1), lambda i: (0, 0)),
            pl.BlockSpec((1, 1), lambda i: (0, 0)),
        ],
        out_specs=pl.BlockSpec((1, blk), lambda i: (0, i)),
        out_shape=jax.ShapeDtypeStruct((1, b_total), jnp.float32),
    )(embT, W1T, b1.reshape(h, 1), W2T.reshape(h, 1), b2.reshape(1, 1))


def kernel(states, emb_table, W1, b1, W2, b2):
    b_total = states.shape[0]
    d = emb_table.shape[1]
    idx1d = states.reshape(b_total)
    tableT = emb_table.T

    iota = lax.iota(jnp.int32, b_total)
    idx_sorted, pos_sorted = lax.sort_key_val(idx1d, iota)
    bnd = jnp.minimum(jnp.arange(_NW + 1) * (_PER_T * _CW), _V).astype(
        jnp.int32).at[_NW].set(_V)
    st = jnp.searchsorted(idx_sorted, bnd, side="left").astype(jnp.int32)
    rng_w = jnp.pad(jnp.stack([st[:-1], st[1:]], axis=1), ((0, 0), (0, 14)))

    flat = _sc_stream_extract(tableT, idx_sorted, pos_sorted, rng_w)
    emb = flat[: b_total * d].reshape(b_total, d)
    valT = _tc_mlp_t(emb.T, W1.T, b1, W2.reshape(-1), b2)
    return (emb, valT.T)
